# baseline (speedup 1.0000x reference)
import jax
import jax.numpy as jnp
from jax import lax
from jax.experimental import pallas as pl
from jax.experimental.pallas import tpu as pltpu
from jax.experimental.pallas import tpu_sc as plsc

N = 10000
E = 320000
F = 128
H = 256
BN_EPS = 1e-5
ALPHA = 0.01

NC = 2
NS = 16
NW = NC * NS
EPW = E // NW
CH = 80
NCHUNK = EPW // CH
NPAIR = (NCHUNK + 1) // 2
ZB = 80
NZC = N // ZB
NF = F // 16


def _rne_bf16(v):
    c = v * jnp.float32(65537.0)
    return c - (c - v)


def _sc_body(x_hbm, ip_hbm, ap_hbm, we_hbm, out_hbm,
             ip0, ip1, ip2, ap0, ap1, ap2, rows0, rows1, rows2, wev, agg,
             sg0, sg1, sg2, si0, si1, si2, ss0, ss1, ss2):
    cid = lax.axis_index("c")
    sid = lax.axis_index("s")
    wid = sid * NC + cid

    zv = jnp.zeros((16,), jnp.float32)

    for i in range(ZB):
        for f in range(NF):
            rows0[i, pl.ds(f * 16, 16)] = zv

    def zchunk(k, carry):
        c = sid + k * NS

        @pl.when(c < NZC)
        def _():
            pltpu.sync_copy(rows0, agg.at[pl.ds(c * ZB, ZB)])

        return carry

    lax.fori_loop(0, (NZC + NS - 1) // NS, zchunk, 0)

    pltpu.sync_copy(we_hbm, wev)
    w0 = [_rne_bf16(wev[0, pl.ds(f * 16, 16)]) for f in range(NF)]
    w1 = [_rne_bf16(wev[1, pl.ds(f * 16, 16)]) for f in range(NF)]

    plsc.subcore_barrier()

    ibase = wid * NCHUNK

    def idx_load(c, ip, ap, si):
        pltpu.async_copy(ip_hbm.at[ibase + c], ip, si)
        pltpu.async_copy(ap_hbm.at[ibase + c], ap, si)

    def idx_wait(c, ip, ap, si):
        pltpu.make_async_copy(ip_hbm.at[ibase + c], ip, si).wait()
        pltpu.make_async_copy(ap_hbm.at[ibase + c], ap, si).wait()

    def gather(c, ip, buf, sg):
        pltpu.async_copy(x_hbm.at[ip.at[0]], buf, sg)

    def gather_wait(c, ip, buf, sg):
        pltpu.make_async_copy(x_hbm.at[ip.at[0]], buf, sg).wait()

    def compute(ap, rows):
        for g in range(CH // 16):
            gb = g * 16
            a0g = _rne_bf16(ap[0, pl.ds(gb, 16)])
            a1g = _rne_bf16(ap[1, pl.ds(gb, 16)])
            for i in range(16):
                s0 = a0g[i]
                s1 = a1g[i]
                e = gb + i
                for f in range(NF):
                    sl = pl.ds(f * 16, 16)
                    v = rows[e, sl] + (s0 * w0[f] + s1 * w1[f])
                    rows[e, sl] = jnp.maximum(v, 0.0)

    bufs = ((ip0, ap0, rows0, sg0, si0, ss0),
            (ip1, ap1, rows1, sg1, si1, ss1),
            (ip2, ap2, rows2, sg2, si2, ss2))

    def slot(c, cur, nxt, wait_guard):
        ip, ap, rows, sg, _, ss = cur
        nip, nap, nrows, nsg, nsi, nss = nxt
        gather_wait(c, ip, rows, sg)
        compute(ap, rows)
        pltpu.async_copy(rows, agg.at[ip.at[1]], ss, add=True)

        @pl.when(c + 2 < NCHUNK)
        def _():
            def drain():
                pltpu.make_async_copy(nrows, agg.at[nip.at[1]], nss).wait()

            if wait_guard is None:
                drain()
            else:
                pl.when(wait_guard)(drain)

            idx_load(c + 2, nip, nap, nsi)
            idx_wait(c + 2, nip, nap, nsi)
            gather(c + 2, nip, nrows, nsg)

    idx_load(0, ip0, ap0, si0)
    idx_load(1, ip1, ap1, si1)
    idx_wait(0, ip0, ap0, si0)
    gather(0, ip0, rows0, sg0)
    idx_wait(1, ip1, ap1, si1)
    gather(1, ip1, rows1, sg1)

    def tri_body(p, carry):
        c = p * 3
        slot(c, bufs[0], bufs[2], wait_guard=(p > 0))

        @pl.when(c + 1 < NCHUNK)
        def _():
            slot(c + 1, bufs[1], bufs[0], wait_guard=None)

        @pl.when(c + 2 < NCHUNK)
        def _():
            slot(c + 2, bufs[2], bufs[1], wait_guard=None)

        return carry

    lax.fori_loop(0, (NCHUNK + 2) // 3, tri_body, 0)

    for ip, _, rows, _, _, ss in bufs:
        pltpu.make_async_copy(rows, agg.at[ip.at[1]], ss).wait()

    plsc.subcore_barrier()

    def wchunk(k, carry):
        c = sid + k * NS

        @pl.when(c < NZC)
        def _():
            pltpu.sync_copy(agg.at[pl.ds(c * ZB, ZB)],
                            out_hbm.at[cid, pl.ds(c * ZB, ZB)])

        return carry

    lax.fori_loop(0, (NZC + NS - 1) // NS, wchunk, 0)


@jax.jit
def _sc_aggregate(xb, ip, ap, We):
    mesh = plsc.VectorSubcoreMesh(core_axis_name="c", subcore_axis_name="s")
    run = pl.kernel(
        _sc_body,
        out_type=jax.ShapeDtypeStruct((NC, N, F), jnp.float32),
        mesh=mesh,
        scratch_types=(
            [pltpu.VMEM((2, CH), jnp.int32)] * 3
            + [pltpu.VMEM((2, CH), jnp.float32)] * 3
            + [pltpu.VMEM((CH, F), jnp.float32)] * 3
            + [pltpu.VMEM((2, F), jnp.float32),
               pltpu.VMEM_SHARED((N, F), jnp.float32)]
            + [pltpu.SemaphoreType.DMA] * 9
        ),
    )
    return run(xb, ip, ap, We)


def _bn_leaky(h, g, b):
    m = jnp.mean(h, axis=0, keepdims=True)
    v = jnp.mean((h - m) * (h - m), axis=0, keepdims=True)
    h = (h - m) / jnp.sqrt(v + BN_EPS) * g + b
    return jnp.where(h >= 0.0, h, ALPHA * h)


def _tc_body(eps_ref, x_ref, p_ref, w1_ref, b1_ref, g1_ref, bt1_ref,
             w2_ref, b2_ref, g2_ref, bt2_ref, gn_ref, btn_ref, out_ref):
    h = (1.0 + eps_ref[0, 0]) * x_ref[...] + p_ref[0] + p_ref[1]
    h = jnp.dot(h, w1_ref[...], preferred_element_type=jnp.float32) + b1_ref[...]
    h = _bn_leaky(h, g1_ref[...], bt1_ref[...])
    h = jnp.dot(h, w2_ref[...], preferred_element_type=jnp.float32) + b2_ref[...]
    h = _bn_leaky(h, g2_ref[...], bt2_ref[...])
    out_ref[...] = _bn_leaky(h, gn_ref[...], btn_ref[...])


@jax.jit
def _tc_mlp(eps, x, p, w1, b1, g1, bt1, w2, b2, g2, bt2, gn, btn):
    return pl.pallas_call(
        _tc_body,
        out_shape=jax.ShapeDtypeStruct((N, H), jnp.float32),
    )(eps, x, p, w1, b1, g1, bt1, w2, b2, g2, bt2, gn, btn)


def kernel(x, edge_idx, edge_attr, We, be, W1, b1, g1, bt1, W2, b2, g2,
           bt2, gn, btn, eps):
    ip = jnp.stack([edge_idx[0].reshape(NW * NCHUNK, CH),
                    edge_idx[1].reshape(NW * NCHUNK, CH)], axis=1)
    ap = jnp.stack([edge_attr[:, 0].reshape(NW * NCHUNK, CH),
                    edge_attr[:, 1].reshape(NW * NCHUNK, CH)], axis=1)
    xb = x + be[None, :]
    partials = _sc_aggregate(xb, ip, ap, We)
    eps_arr = jnp.reshape(eps, (1, 1)).astype(jnp.float32)
    return _tc_mlp(eps_arr, x, partials,
                   W1, b1[None, :], g1[None, :], bt1[None, :],
                   W2, b2[None, :], g2[None, :], bt2[None, :],
                   gn[None, :], btn[None, :])


# speedup vs baseline: 1.6011x; 1.6011x over previous
import jax
import jax.numpy as jnp
from jax import lax
from jax.experimental import pallas as pl
from jax.experimental.pallas import tpu as pltpu
from jax.experimental.pallas import tpu_sc as plsc

N = 10000
E = 320000
F = 128
H = 256
BN_EPS = 1e-5
ALPHA = 0.01

NC = 2
NS = 16
NW = NC * NS
EPW = E // NW
CH = 80
NCHUNK = EPW // CH
NPAIR = (NCHUNK + 1) // 2
ZB = 80
NZC = N // ZB
NF = F // 16


def _rne_bf16(v):
    c = v * jnp.float32(65537.0)
    return c - (c - v)


def _sc_body(x_hbm, ip_hbm, ap_hbm, we_hbm, out_hbm,
             ip0, ip1, ap0, ap1, ipp, app, rows0, rows1, wev, agg,
             sg0, sg1, si0, si1, sip):
    cid = lax.axis_index("c")
    sid = lax.axis_index("s")
    wid = sid * NC + cid

    zv = jnp.zeros((16,), jnp.float32)

    for i in range(ZB):
        for f in range(NF):
            rows0[i, pl.ds(f * 16, 16)] = zv

    def zchunk(k, carry):
        c = sid + k * NS

        @pl.when(c < NZC)
        def _():
            pltpu.sync_copy(rows0, agg.at[pl.ds(c * ZB, ZB)])

        return carry

    lax.fori_loop(0, (NZC + NS - 1) // NS, zchunk, 0)

    pltpu.sync_copy(we_hbm, wev)
    w0 = [_rne_bf16(wev[0, pl.ds(f * 16, 16)]) for f in range(NF)]
    w1 = [_rne_bf16(wev[1, pl.ds(f * 16, 16)]) for f in range(NF)]

    plsc.subcore_barrier()

    ibase = wid * NCHUNK

    def idx_load(c, ip, ap, si):
        pltpu.async_copy(ip_hbm.at[ibase + c], ip, si)
        pltpu.async_copy(ap_hbm.at[ibase + c], ap, si)

    def idx_wait(c, ip, ap, si):
        pltpu.make_async_copy(ip_hbm.at[ibase + c], ip, si).wait()
        pltpu.make_async_copy(ap_hbm.at[ibase + c], ap, si).wait()

    def gather(c, ip, buf, sg):
        pltpu.async_copy(x_hbm.at[ip.at[0]], buf, sg)

    def gather_wait(c, ip, buf, sg):
        pltpu.make_async_copy(x_hbm.at[ip.at[0]], buf, sg).wait()

    def compute_scatter(ip, ap, rows):
        for g in range(CH // 16):
            gb = g * 16
            a0g = _rne_bf16(ap[0, pl.ds(gb, 16)])
            a1g = _rne_bf16(ap[1, pl.ds(gb, 16)])
            for i in range(16):
                s0 = a0g[i]
                s1 = a1g[i]
                e = gb + i
                for f in range(NF):
                    sl = pl.ds(f * 16, 16)
                    v = rows[e, sl] + (s0 * w0[f] + s1 * w1[f])
                    rows[e, sl] = jnp.maximum(v, 0.0)

        pltpu.sync_copy(rows, agg.at[ip.at[1]], add=True)

    idx_load(0, ip0, ap0, si0)
    idx_load(1, ip1, ap1, si1)
    idx_wait(0, ip0, ap0, si0)
    gather(0, ip0, rows0, sg0)
    idx_wait(1, ip1, ap1, si1)
    gather(1, ip1, rows1, sg1)

    def commit_idx(ip, ap):
        for r in range(2):
            for k in range(CH // 16):
                sl = pl.ds(k * 16, 16)
                ip[r, sl] = ipp[r, sl]
                ap[r, sl] = app[r, sl]

    def pair_body(p, carry):
        c0 = p * 2
        c1 = c0 + 1

        gather_wait(c0, ip0, rows0, sg0)

        @pl.when(c0 + 2 < NCHUNK)
        def _():
            idx_load(c0 + 2, ipp, app, sip)

        compute_scatter(ip0, ap0, rows0)

        @pl.when(c0 + 2 < NCHUNK)
        def _():
            idx_wait(c0 + 2, ipp, app, sip)
            commit_idx(ip0, ap0)
            gather(c0 + 2, ip0, rows0, sg0)

        @pl.when(c1 < NCHUNK)
        def _():
            gather_wait(c1, ip1, rows1, sg1)

            @pl.when(c1 + 2 < NCHUNK)
            def _():
                idx_load(c1 + 2, ipp, app, sip)

            compute_scatter(ip1, ap1, rows1)

            @pl.when(c1 + 2 < NCHUNK)
            def _():
                idx_wait(c1 + 2, ipp, app, sip)
                commit_idx(ip1, ap1)
                gather(c1 + 2, ip1, rows1, sg1)

        return carry

    lax.fori_loop(0, NPAIR, pair_body, 0)

    plsc.subcore_barrier()

    def wchunk(k, carry):
        c = sid + k * NS

        @pl.when(c < NZC)
        def _():
            pltpu.sync_copy(agg.at[pl.ds(c * ZB, ZB)],
                            out_hbm.at[cid, pl.ds(c * ZB, ZB)])

        return carry

    lax.fori_loop(0, (NZC + NS - 1) // NS, wchunk, 0)


@jax.jit
def _sc_aggregate(xb, ip, ap, We):
    mesh = plsc.VectorSubcoreMesh(core_axis_name="c", subcore_axis_name="s")
    run = pl.kernel(
        _sc_body,
        out_type=jax.ShapeDtypeStruct((NC, N, F), jnp.float32),
        mesh=mesh,
        scratch_types=[
            pltpu.VMEM((2, CH), jnp.int32),
            pltpu.VMEM((2, CH), jnp.int32),
            pltpu.VMEM((2, CH), jnp.float32),
            pltpu.VMEM((2, CH), jnp.float32),
            pltpu.VMEM((2, CH), jnp.int32),
            pltpu.VMEM((2, CH), jnp.float32),
            pltpu.VMEM((CH, F), jnp.float32),
            pltpu.VMEM((CH, F), jnp.float32),
            pltpu.VMEM((2, F), jnp.float32),
            pltpu.VMEM_SHARED((N, F), jnp.float32),
            pltpu.SemaphoreType.DMA,
            pltpu.SemaphoreType.DMA,
            pltpu.SemaphoreType.DMA,
            pltpu.SemaphoreType.DMA,
            pltpu.SemaphoreType.DMA,
        ],
    )
    return run(xb, ip, ap, We)


def _bn_leaky(h, g, b):
    m = jnp.mean(h, axis=0, keepdims=True)
    v = jnp.mean((h - m) * (h - m), axis=0, keepdims=True)
    h = (h - m) / jnp.sqrt(v + BN_EPS) * g + b
    return jnp.where(h >= 0.0, h, ALPHA * h)


def _tc_body(eps_ref, x_ref, p_ref, w1_ref, b1_ref, g1_ref, bt1_ref,
             w2_ref, b2_ref, g2_ref, bt2_ref, gn_ref, btn_ref, out_ref):
    h = (1.0 + eps_ref[0, 0]) * x_ref[...] + p_ref[0] + p_ref[1]
    h = jnp.dot(h, w1_ref[...], preferred_element_type=jnp.float32) + b1_ref[...]
    h = _bn_leaky(h, g1_ref[...], bt1_ref[...])
    h = jnp.dot(h, w2_ref[...], preferred_element_type=jnp.float32) + b2_ref[...]
    h = _bn_leaky(h, g2_ref[...], bt2_ref[...])
    out_ref[...] = _bn_leaky(h, gn_ref[...], btn_ref[...])


@jax.jit
def _tc_mlp(eps, x, p, w1, b1, g1, bt1, w2, b2, g2, bt2, gn, btn):
    return pl.pallas_call(
        _tc_body,
        out_shape=jax.ShapeDtypeStruct((N, H), jnp.float32),
    )(eps, x, p, w1, b1, g1, bt1, w2, b2, g2, bt2, gn, btn)


def kernel(x, edge_idx, edge_attr, We, be, W1, b1, g1, bt1, W2, b2, g2,
           bt2, gn, btn, eps):
    ip = jnp.stack([edge_idx[0].reshape(NW * NCHUNK, CH),
                    edge_idx[1].reshape(NW * NCHUNK, CH)], axis=1)
    ap = jnp.stack([edge_attr[:, 0].reshape(NW * NCHUNK, CH),
                    edge_attr[:, 1].reshape(NW * NCHUNK, CH)], axis=1)
    xb = x + be[None, :]
    partials = _sc_aggregate(xb, ip, ap, We)
    eps_arr = jnp.reshape(eps, (1, 1)).astype(jnp.float32)
    return _tc_mlp(eps_arr, x, partials,
                   W1, b1[None, :], g1[None, :], bt1[None, :],
                   W2, b2[None, :], g2[None, :], bt2[None, :],
                   gn[None, :], btn[None, :])
